# baseline (speedup 1.0000x reference)
import functools

import jax
import jax.numpy as jnp
from jax import lax
from jax.experimental import pallas as pl
from jax.experimental.pallas import tpu as pltpu
from jax.experimental.pallas import tpu_sc as plsc

_PENALTY_WEIGHT = 1.0
_MIN_ENV_SAMPLES = 2.0
_E = 8
_SC_TAIL = 256


def _make_sc_partial(b, d, tail_base, nc, ns):
    s_w = (b - tail_base) // (nc * ns)
    mesh = plsc.VectorSubcoreMesh(core_axis_name="c", subcore_axis_name="s")

    @functools.partial(
        pl.kernel, mesh=mesh,
        out_type=jax.ShapeDtypeStruct((nc, _E * d, d), jnp.float32),
        scratch_types=[
            pltpu.VMEM((s_w, d), jnp.int32),
            pltpu.VMEM((d, d), jnp.float32),
            pltpu.VMEM_SHARED((_E * d, d), jnp.float32),
        ],
    )
    def sc_partial(a_hbm, dest_hbm, zeros_hbm, out_hbm,
                   idx_v, slab_v, acc_sh):
        c = lax.axis_index("c")
        s = lax.axis_index("s")
        my_base = tail_base + (c * ns + s) * s_w

        @pl.when(s == 0)
        def _zero():
            pltpu.sync_copy(zeros_hbm, acc_sh)

        plsc.subcore_barrier()

        pltpu.sync_copy(dest_hbm.at[pl.ds(my_base, s_w)], idx_v)
        for g in range(s_w):
            pltpu.sync_copy(a_hbm.at[pl.ds((my_base + g) * d, d)], slab_v)
            pltpu.sync_copy(slab_v, acc_sh.at[idx_v.at[g]], add=True)

        plsc.subcore_barrier()

        @pl.when(s == 0)
        def _flush():
            pltpu.sync_copy(acc_sh, out_hbm.at[c])

    return sc_partial


def _make_tc_kernel(b, bb, d, tail_base, nc):
    def _irm_kernel(lab_ref, a_ref, sc_ref, out_ref, acc_ref, cnt_ref):
        i = pl.program_id(0)
        n = pl.num_programs(0)

        @pl.when(i == 0)
        def _init():
            acc_ref[...] = jnp.zeros_like(acc_ref)
            for e in range(_E):
                cnt_ref[0, e] = 0.0

        def _body(s, carry):
            lab = lab_ref[0, i * bb + s]
            acc_ref[pl.ds(lab * d, d), :] += a_ref[pl.ds(s * d, d), :]
            cnt_ref[0, lab] += 1.0
            return carry

        jax.lax.fori_loop(0, bb, _body, 0, unroll=True)

        @pl.when(i == n - 1)
        def _finish():
            def _cbody(t, carry):
                lab = lab_ref[0, tail_base + t]
                cnt_ref[0, lab] += 1.0
                return carry

            jax.lax.fori_loop(0, b - tail_base, _cbody, 0)

            counts = [cnt_ref[0, e] for e in range(_E)]
            valid = [jnp.where(c >= _MIN_ENV_SAMPLES, 1.0, 0.0) for c in counts]
            safe = [jnp.maximum(c, 1.0) for c in counts]
            n_valid = sum(valid)

            def _sums(e):
                r = acc_ref[e * d:(e + 1) * d, :]
                for c in range(nc):
                    r = r + sc_ref[c, e * d:(e + 1) * d, :]
                return r

            mom = jnp.zeros((d, d), jnp.float32)
            for e in range(_E):
                mom += (valid[e] / (safe[e] * n_valid)) * _sums(e)
            var = jnp.zeros((d, d), jnp.float32)
            for e in range(_E):
                diff = _sums(e) / safe[e] - mom
                var += valid[e] * diff * diff
            out_ref[0, 0] = jnp.sum(var) / (n_valid - 1.0)

    return _irm_kernel


def kernel(A_batch, env_labels):
    b, d, _ = A_batch.shape
    a2 = A_batch.reshape(b * d, d)
    labs32 = env_labels.astype(jnp.int32)

    info = plsc.get_sparse_core_info()
    nc, ns = info.num_cores, info.num_subcores
    tail_base = b - _SC_TAIL
    zeros = jnp.zeros((_E * d, d), jnp.float32)
    dest = labs32[:, None] * d + jnp.arange(d, dtype=jnp.int32)[None, :]
    sc_partial = _make_sc_partial(b, d, tail_base, nc, ns)(a2, dest, zeros)

    labs = labs32.reshape(1, b)
    bb = 128
    g = tail_base // bb
    out = pl.pallas_call(
        _make_tc_kernel(b, bb, d, tail_base, nc),
        grid=(g,),
        in_specs=[
            pl.BlockSpec(memory_space=pltpu.SMEM),
            pl.BlockSpec((bb * d, d), lambda i: (i, 0)),
            pl.BlockSpec((nc, _E * d, d), lambda i: (0, 0, 0)),
        ],
        out_specs=pl.BlockSpec((1, 1), lambda i: (0, 0),
                               memory_space=pltpu.SMEM),
        out_shape=jax.ShapeDtypeStruct((1, 1), jnp.float32),
        scratch_shapes=[
            pltpu.VMEM((_E * d, d), jnp.float32),
            pltpu.SMEM((1, _E), jnp.float32),
        ],
        compiler_params=pltpu.CompilerParams(
            dimension_semantics=("arbitrary",),
        ),
    )(labs, a2, sc_partial)
    return out[0, 0] * (_PENALTY_WEIGHT / (d * d))


# speedup vs baseline: 1.1303x; 1.1303x over previous
import functools

import jax
import jax.numpy as jnp
from jax import lax
from jax.experimental import pallas as pl
from jax.experimental.pallas import tpu as pltpu
from jax.experimental.pallas import tpu_sc as plsc

_PENALTY_WEIGHT = 1.0
_MIN_ENV_SAMPLES = 2.0
_E = 8
_SC_TAIL = 256


def _make_sc_partial(b, d, tail_base, nc, ns):
    s_w = (b - tail_base) // (nc * ns)

    @functools.partial(
        pl.kernel,
        mesh=plsc.VectorSubcoreMesh(core_axis_name="c", subcore_axis_name="s"),
        out_type=jax.ShapeDtypeStruct((nc, _E * d, d), jnp.float32),
        scratch_types=[
            pltpu.VMEM((s_w, d), jnp.int32),
            pltpu.VMEM((2, d, d), jnp.float32),
            pltpu.VMEM_SHARED((_E * d, d), jnp.float32),
            pltpu.SemaphoreType.DMA,
            pltpu.SemaphoreType.DMA,
        ],
    )
    def sc_partial(a_hbm, dest_hbm, zeros_hbm, out_hbm,
                   idx_v, slab_v, acc_sh, sem0, sem1):
        c = lax.axis_index("c")
        s = lax.axis_index("s")
        my_base = tail_base + (c * ns + s) * s_w
        sems = [sem0, sem1]

        @pl.when(s == 0)
        def _zero():
            pltpu.sync_copy(zeros_hbm, acc_sh)

        plsc.subcore_barrier()

        pltpu.sync_copy(dest_hbm.at[pl.ds(my_base, s_w)], idx_v)
        pending = pltpu.async_copy(
            a_hbm.at[pl.ds(my_base * d, d)], slab_v.at[0], sems[0])
        for g in range(s_w):
            nxt = None
            if g + 1 < s_w:
                nxt = pltpu.async_copy(
                    a_hbm.at[pl.ds((my_base + g + 1) * d, d)],
                    slab_v.at[(g + 1) % 2], sems[(g + 1) % 2])
            pending.wait()
            pltpu.sync_copy(slab_v.at[g % 2], acc_sh.at[idx_v.at[g]], add=True)
            pending = nxt

        plsc.subcore_barrier()

        @pl.when(s == 0)
        def _flush():
            pltpu.sync_copy(acc_sh, out_hbm.at[c])

    return sc_partial


def _make_tc_partial(bb, d):
    def _tc_partial(lab_ref, a_ref, acc_ref):
        i = pl.program_id(0)

        @pl.when(i == 0)
        def _init():
            acc_ref[...] = jnp.zeros_like(acc_ref)

        def _body(s, carry):
            lab = lab_ref[0, i * bb + s]
            acc_ref[pl.ds(lab * d, d), :] += a_ref[pl.ds(s * d, d), :]
            return carry

        jax.lax.fori_loop(0, bb, _body, 0, unroll=True)

    return _tc_partial


def _make_finisher(b, d, nc):
    def _finisher(lab_ref, tc_ref, sc_ref, out_ref, cnt_ref):
        for e in range(_E):
            cnt_ref[0, e] = 0.0

        def _cbody(t, carry):
            lab = lab_ref[0, t]
            cnt_ref[0, lab] += 1.0
            return carry

        jax.lax.fori_loop(0, b, _cbody, 0)

        counts = [cnt_ref[0, e] for e in range(_E)]
        valid = [jnp.where(c >= _MIN_ENV_SAMPLES, 1.0, 0.0) for c in counts]
        safe = [jnp.maximum(c, 1.0) for c in counts]
        n_valid = sum(valid)

        def _sums(e):
            r = tc_ref[e * d:(e + 1) * d, :]
            for c in range(nc):
                r = r + sc_ref[c, e * d:(e + 1) * d, :]
            return r

        mom = jnp.zeros((d, d), jnp.float32)
        for e in range(_E):
            mom += (valid[e] / (safe[e] * n_valid)) * _sums(e)
        var = jnp.zeros((d, d), jnp.float32)
        for e in range(_E):
            diff = _sums(e) / safe[e] - mom
            var += valid[e] * diff * diff
        out_ref[0, 0] = jnp.sum(var) / (n_valid - 1.0)

    return _finisher


def kernel(A_batch, env_labels):
    b, d, _ = A_batch.shape
    a2 = A_batch.reshape(b * d, d)
    labs32 = env_labels.astype(jnp.int32)
    labs = labs32.reshape(1, b)

    info = plsc.get_sparse_core_info()
    nc, ns = info.num_cores, info.num_subcores
    tail_base = b - _SC_TAIL
    zeros = jnp.zeros((_E * d, d), jnp.float32)
    dest = labs32[:, None] * d + jnp.arange(d, dtype=jnp.int32)[None, :]
    sc_acc = _make_sc_partial(b, d, tail_base, nc, ns)(a2, dest, zeros)

    bb = 128
    g = tail_base // bb
    tc_acc = pl.pallas_call(
        _make_tc_partial(bb, d),
        grid=(g,),
        in_specs=[
            pl.BlockSpec(memory_space=pltpu.SMEM),
            pl.BlockSpec((bb * d, d), lambda i: (i, 0)),
        ],
        out_specs=pl.BlockSpec((_E * d, d), lambda i: (0, 0)),
        out_shape=jax.ShapeDtypeStruct((_E * d, d), jnp.float32),
        compiler_params=pltpu.CompilerParams(
            dimension_semantics=("arbitrary",),
        ),
    )(labs, a2)

    out = pl.pallas_call(
        _make_finisher(b, d, nc),
        in_specs=[
            pl.BlockSpec(memory_space=pltpu.SMEM),
            pl.BlockSpec((_E * d, d), lambda: (0, 0)),
            pl.BlockSpec((nc, _E * d, d), lambda: (0, 0, 0)),
        ],
        out_specs=pl.BlockSpec((1, 1), lambda: (0, 0),
                               memory_space=pltpu.SMEM),
        out_shape=jax.ShapeDtypeStruct((1, 1), jnp.float32),
        scratch_shapes=[pltpu.SMEM((1, _E), jnp.float32)],
    )(labs, tc_acc, sc_acc)
    return out[0, 0] * (_PENALTY_WEIGHT / (d * d))
